# baseline (speedup 1.0000x reference)
import jax
import jax.numpy as jnp
from jax.experimental import pallas as pl
from jax.experimental.pallas import tpu as pltpu

B, N, C = 8, 1024, 384
K = 1024
N_ITER = 6
T = B * N
TB = 1024
NBLK = T // TB


def _norm_rows(v):
    n = jnp.sqrt(jnp.sum(v * v, axis=-1, keepdims=True))
    return v / jnp.maximum(n, 1e-12)


def _cam_kernel(x_ref, means_ref, out_ref, m_ref, sums_ref, cnt_ref):
    m_ref[...] = _norm_rows(means_ref[...])
    for b in range(NBLK):
        out_ref[pl.ds(b * TB, TB), :] = _norm_rows(x_ref[pl.ds(b * TB, TB), :])

    iota_row = jax.lax.broadcasted_iota(jnp.int32, (1, K), 1)
    ones_tb = jnp.ones((TB, 128), dtype=jnp.float32)

    def assign_block(b):
        xb = out_ref[pl.ds(b * TB, TB), :]
        d = jax.lax.dot_general(
            xb, m_ref[...], (((1,), (1,)), ((), ())),
            preferred_element_type=jnp.float32)
        idx = jnp.argmax(d, axis=1, keepdims=True)
        oh = (iota_row == idx).astype(jnp.float32)
        return xb, oh

    def refine_iter(_, carry):
        sums_ref[...] = jnp.zeros_like(sums_ref)
        cnt_ref[...] = jnp.zeros_like(cnt_ref)

        def block_body(b, carry2):
            xb, oh = assign_block(b)
            sums_ref[...] += jax.lax.dot_general(
                oh, xb, (((0,), (0,)), ((), ())),
                preferred_element_type=jnp.float32)
            cnt_ref[...] += jax.lax.dot_general(
                oh, ones_tb, (((0,), (0,)), ((), ())),
                preferred_element_type=jnp.float32)
            return carry2

        jax.lax.fori_loop(0, NBLK, block_body, 0)
        counts = cnt_ref[:, 0:1]
        mn = _norm_rows(sums_ref[...] / jnp.maximum(counts, 1.0))
        m_ref[...] = jnp.where(counts == 0.0, m_ref[...], mn)
        return carry

    jax.lax.fori_loop(0, N_ITER - 1, refine_iter, 0)

    def out_block(b, carry):
        _, oh = assign_block(b)
        q = jax.lax.dot_general(
            oh, m_ref[...], (((1,), (0,)), ((), ())),
            preferred_element_type=jnp.float32)
        xraw = x_ref[pl.ds(b * TB, TB), :]
        out_ref[pl.ds(b * TB, TB), :] = xraw + (q - xraw)
        return carry

    jax.lax.fori_loop(0, NBLK, out_block, 0)


@jax.jit
def kernel(x, means):
    xf = x.reshape(T, C)
    out = pl.pallas_call(
        _cam_kernel,
        out_shape=jax.ShapeDtypeStruct((T, C), jnp.float32),
        scratch_shapes=[
            pltpu.VMEM((K, C), jnp.float32),
            pltpu.VMEM((K, C), jnp.float32),
            pltpu.VMEM((K, 128), jnp.float32),
        ],
    )(xf, means)
    return out.reshape(B, N, C)


# speedup vs baseline: 1.2720x; 1.2720x over previous
import jax
import jax.numpy as jnp
from jax.experimental import pallas as pl
from jax.experimental.pallas import tpu as pltpu

B, N, C = 8, 1024, 384
K = 1024
N_ITER = 6
T = B * N
TB = 1024
NBLK = T // TB
EPS_EMPTY = 1e-20


def _norm_rows(v):
    n = jnp.sqrt(jnp.sum(v * v, axis=-1, keepdims=True))
    return v / jnp.maximum(n, 1e-12)


def _cam_kernel(x_ref, means_ref, out_ref, m_ref, sums_ref):
    m_ref[...] = _norm_rows(means_ref[...])
    lane_iota = jax.lax.broadcasted_iota(jnp.int32, (TB, K), 1)

    def assign_block(b):
        xb = _norm_rows(x_ref[pl.ds(b * TB, TB), :])
        d = jax.lax.dot_general(
            xb, m_ref[...], (((1,), (1,)), ((), ())),
            preferred_element_type=jnp.float32)
        maxv = jnp.max(d, axis=1, keepdims=True)
        idx = jnp.min(jnp.where(d == maxv, lane_iota, K), axis=1,
                      keepdims=True)
        oh = (lane_iota == idx).astype(jnp.float32)
        return xb, oh

    def refine_iter(_, carry):
        sums_ref[...] = EPS_EMPTY * m_ref[...]

        def block_body(b, carry2):
            xb, oh = assign_block(b)
            sums_ref[...] += jax.lax.dot_general(
                oh, xb, (((0,), (0,)), ((), ())),
                preferred_element_type=jnp.float32)
            return carry2

        jax.lax.fori_loop(0, NBLK, block_body, 0)
        m_ref[...] = _norm_rows(sums_ref[...])
        return carry

    jax.lax.fori_loop(0, N_ITER - 1, refine_iter, 0)

    def out_block(b, carry):
        _, oh = assign_block(b)
        q = jax.lax.dot_general(
            oh, m_ref[...], (((1,), (0,)), ((), ())),
            preferred_element_type=jnp.float32)
        xraw = x_ref[pl.ds(b * TB, TB), :]
        out_ref[pl.ds(b * TB, TB), :] = xraw + (q - xraw)
        return carry

    jax.lax.fori_loop(0, NBLK, out_block, 0)


@jax.jit
def kernel(x, means):
    xf = x.reshape(T, C)
    out = pl.pallas_call(
        _cam_kernel,
        out_shape=jax.ShapeDtypeStruct((T, C), jnp.float32),
        scratch_shapes=[
            pltpu.VMEM((K, C), jnp.float32),
            pltpu.VMEM((K, C), jnp.float32),
        ],
    )(xf, means)
    return out.reshape(B, N, C)


# speedup vs baseline: 1.2907x; 1.0147x over previous
import jax
import jax.numpy as jnp
from jax.experimental import pallas as pl
from jax.experimental.pallas import tpu as pltpu

B, N, C = 8, 1024, 384
K = 1024
N_ITER = 6
T = B * N
TB = 1024
NBLK = T // TB
EPS_EMPTY = 1e-20


def _norm_rows(v):
    n = jnp.sqrt(jnp.sum(v * v, axis=-1, keepdims=True))
    return v / jnp.maximum(n, 1e-12)


def _cam_kernel(x_ref, means_ref, out_ref, m_ref, sums_ref):
    m_ref[...] = _norm_rows(means_ref[...])
    lane_iota = jax.lax.broadcasted_iota(jnp.int32, (TB, K), 1)

    def norm_block(b, carry):
        out_ref[pl.ds(b * TB, TB), :] = _norm_rows(x_ref[pl.ds(b * TB, TB), :])
        return carry

    jax.lax.fori_loop(0, NBLK, norm_block, 0)

    def assign_block(b):
        xb = out_ref[pl.ds(b * TB, TB), :]
        d = jax.lax.dot_general(
            xb, m_ref[...], (((1,), (1,)), ((), ())),
            preferred_element_type=jnp.float32)
        maxv = jnp.max(d, axis=1, keepdims=True)
        idx = jnp.min(jnp.where(d == maxv, lane_iota, K), axis=1,
                      keepdims=True)
        oh = (lane_iota == idx).astype(jnp.float32)
        return xb, oh

    def refine_iter(_, carry):
        sums_ref[...] = EPS_EMPTY * m_ref[...]

        def block_body(b, carry2):
            xb, oh = assign_block(b)
            sums_ref[...] += jax.lax.dot_general(
                oh, xb, (((0,), (0,)), ((), ())),
                preferred_element_type=jnp.float32)
            return carry2

        jax.lax.fori_loop(0, NBLK, block_body, 0)
        m_ref[...] = _norm_rows(sums_ref[...])
        return carry

    jax.lax.fori_loop(0, N_ITER - 1, refine_iter, 0)

    def out_block(b, carry):
        _, oh = assign_block(b)
        q = jax.lax.dot_general(
            oh, m_ref[...], (((1,), (0,)), ((), ())),
            preferred_element_type=jnp.float32)
        xraw = x_ref[pl.ds(b * TB, TB), :]
        out_ref[pl.ds(b * TB, TB), :] = xraw + (q - xraw)
        return carry

    jax.lax.fori_loop(0, NBLK, out_block, 0)


@jax.jit
def kernel(x, means):
    xf = x.reshape(T, C)
    out = pl.pallas_call(
        _cam_kernel,
        out_shape=jax.ShapeDtypeStruct((T, C), jnp.float32),
        scratch_shapes=[
            pltpu.VMEM((K, C), jnp.float32),
            pltpu.VMEM((K, C), jnp.float32),
        ],
    )(xf, means)
    return out.reshape(B, N, C)


# speedup vs baseline: 1.3816x; 1.0704x over previous
import jax
import jax.numpy as jnp
from jax.experimental import pallas as pl
from jax.experimental.pallas import tpu as pltpu

B, N, C = 8, 1024, 384
K = 1024
N_ITER = 6
T = B * N
TB = 2048
NBLK = T // TB
EPS_EMPTY = 1e-20


def _norm_rows(v):
    n = jnp.sqrt(jnp.sum(v * v, axis=-1, keepdims=True))
    return v / jnp.maximum(n, 1e-12)


def _cam_kernel(x_ref, means_ref, out_ref, m_ref, sums_ref):
    m_ref[...] = _norm_rows(means_ref[...])
    lane_iota = jax.lax.broadcasted_iota(jnp.int32, (1, K), 1)

    def norm_block(b, carry):
        out_ref[pl.ds(b * TB, TB), :] = _norm_rows(x_ref[pl.ds(b * TB, TB), :])
        return carry

    jax.lax.fori_loop(0, NBLK, norm_block, 0)

    def assign_block(b):
        xb = out_ref[pl.ds(b * TB, TB), :]
        d = jax.lax.dot_general(
            xb, m_ref[...], (((1,), (1,)), ((), ())),
            preferred_element_type=jnp.float32)
        maxv = jnp.max(d, axis=1, keepdims=True)
        idx = jnp.min(jnp.where(d == maxv, lane_iota, K), axis=1,
                      keepdims=True)
        oh = (lane_iota == idx).astype(jnp.float32)
        return xb, oh

    def refine_iter(_, carry):
        sums_ref[...] = EPS_EMPTY * m_ref[...]

        def block_body(b, carry2):
            xb, oh = assign_block(b)
            sums_ref[...] += jax.lax.dot_general(
                oh, xb, (((0,), (0,)), ((), ())),
                preferred_element_type=jnp.float32)
            return carry2

        jax.lax.fori_loop(0, NBLK, block_body, 0)
        m_ref[...] = _norm_rows(sums_ref[...])
        return carry

    jax.lax.fori_loop(0, N_ITER - 1, refine_iter, 0)

    def out_block(b, carry):
        _, oh = assign_block(b)
        q = jax.lax.dot_general(
            oh, m_ref[...], (((1,), (0,)), ((), ())),
            preferred_element_type=jnp.float32)
        xraw = x_ref[pl.ds(b * TB, TB), :]
        out_ref[pl.ds(b * TB, TB), :] = xraw + (q - xraw)
        return carry

    jax.lax.fori_loop(0, NBLK, out_block, 0)


@jax.jit
def kernel(x, means):
    xf = x.reshape(T, C)
    out = pl.pallas_call(
        _cam_kernel,
        out_shape=jax.ShapeDtypeStruct((T, C), jnp.float32),
        scratch_shapes=[
            pltpu.VMEM((K, C), jnp.float32),
            pltpu.VMEM((K, C), jnp.float32),
        ],
    )(xf, means)
    return out.reshape(B, N, C)


# speedup vs baseline: 1.4413x; 1.0432x over previous
import jax
import jax.numpy as jnp
from jax.experimental import pallas as pl
from jax.experimental.pallas import tpu as pltpu

B, N, C = 8, 1024, 384
K = 1024
N_ITER = 6
T = B * N
TB = 4096
NBLK = T // TB
EPS_EMPTY = 1e-20


def _norm_rows(v):
    n = jnp.sqrt(jnp.sum(v * v, axis=-1, keepdims=True))
    return v / jnp.maximum(n, 1e-12)


def _cam_kernel(x_ref, means_ref, out_ref, m_ref, sums_ref):
    m_ref[...] = _norm_rows(means_ref[...])
    lane_iota = jax.lax.broadcasted_iota(jnp.int32, (1, K), 1)

    def norm_block(b, carry):
        out_ref[pl.ds(b * TB, TB), :] = _norm_rows(x_ref[pl.ds(b * TB, TB), :])
        return carry

    jax.lax.fori_loop(0, NBLK, norm_block, 0)

    def assign_block(b):
        xb = out_ref[pl.ds(b * TB, TB), :]
        d = jax.lax.dot_general(
            xb, m_ref[...], (((1,), (1,)), ((), ())),
            preferred_element_type=jnp.float32)
        maxv = jnp.max(d, axis=1, keepdims=True)
        idx = jnp.min(jnp.where(d == maxv, lane_iota, K), axis=1,
                      keepdims=True)
        oh = (lane_iota == idx).astype(jnp.float32)
        return xb, oh

    def refine_iter(_, carry):
        sums_ref[...] = EPS_EMPTY * m_ref[...]

        def block_body(b, carry2):
            xb, oh = assign_block(b)
            sums_ref[...] += jax.lax.dot_general(
                oh, xb, (((0,), (0,)), ((), ())),
                preferred_element_type=jnp.float32)
            return carry2

        jax.lax.fori_loop(0, NBLK, block_body, 0)
        m_ref[...] = _norm_rows(sums_ref[...])
        return carry

    jax.lax.fori_loop(0, N_ITER - 1, refine_iter, 0)

    def out_block(b, carry):
        _, oh = assign_block(b)
        q = jax.lax.dot_general(
            oh, m_ref[...], (((1,), (0,)), ((), ())),
            preferred_element_type=jnp.float32)
        xraw = x_ref[pl.ds(b * TB, TB), :]
        out_ref[pl.ds(b * TB, TB), :] = xraw + (q - xraw)
        return carry

    jax.lax.fori_loop(0, NBLK, out_block, 0)


@jax.jit
def kernel(x, means):
    xf = x.reshape(T, C)
    out = pl.pallas_call(
        _cam_kernel,
        out_shape=jax.ShapeDtypeStruct((T, C), jnp.float32),
        scratch_shapes=[
            pltpu.VMEM((K, C), jnp.float32),
            pltpu.VMEM((K, C), jnp.float32),
        ],
    )(xf, means)
    return out.reshape(B, N, C)
